# baseline (speedup 1.0000x reference)
import functools

import jax
import jax.numpy as jnp
from jax import lax
from jax.experimental import pallas as pl
from jax.experimental.pallas import tpu as pltpu
from jax.experimental.pallas import tpu_sc as plsc

KP1 = 31
NB = 128
NLANE = 16
NWORKERS = 32


def _prep_body(f_ref, w1_ref, w2_ref, emb_ref):
    h = f_ref[...] * w1_ref[...]
    h = jnp.maximum(h, 0.0) * w2_ref[...]
    n2 = jnp.sum(h * h, axis=1, keepdims=True)
    norm = jnp.sqrt(n2)
    emb_ref[...] = h / jnp.maximum(norm, 1e-12)


def _mm_body(a_ref, b_ref, out_ref):
    out_ref[...] = lax.dot_general(
        a_ref[...], b_ref[...], (((1,), (1,)), ((), ())),
        preferred_element_type=jnp.float32)


def _sc_topk_body(sim_ref, out_ref, rowbuf, outbuf, hist, *, ncol):
    nvreg = ncol // NLANE
    lanes = lax.iota(jnp.int32, NLANE)
    lane_lo = (lanes * NB).astype(jnp.float32)
    lane_hi = lane_lo + jnp.float32(NB - 1)
    ones16 = jnp.ones((NLANE,), jnp.float32)
    zeros16 = jnp.zeros((NLANE,), jnp.float32)

    def zero_hist():
        def zbody(j, c):
            hist[pl.ds(j * NLANE, NLANE)] = zeros16
            return c
        lax.fori_loop(0, NB * NLANE // NLANE, zbody, 0)

    def hist_pass(lo, scale):
        off = lane_lo - lo * scale
        def hbody(i, c):
            v = rowbuf[pl.ds(i * NLANE, NLANE)]
            x = v * scale + off
            x = jnp.minimum(jnp.maximum(x, lane_lo), lane_hi)
            plsc.addupdate_scatter(hist, [x.astype(jnp.int32)], ones16)
            return c
        lax.fori_loop(0, nvreg, hbody, 0)

    def find():
        carry = jnp.float32(0.0)
        B = jnp.float32(-1.0)
        S = jnp.float32(0.0)
        for j in reversed(range(NB // NLANE)):
            a = hist[pl.ds(j * NLANE + 0 * NB, NLANE)]
            for l in range(1, NLANE):
                a = a + hist[pl.ds(j * NLANE + l * NB, NLANE)]
            cs = plsc.cumsum(lax.rev(a, (0,))) + carry
            m = cs >= jnp.float32(KP1)
            bidx = jnp.float32(NLANE * j + NLANE - 1) - lax.iota(
                jnp.int32, NLANE).astype(jnp.float32)
            bc = jnp.max(jnp.where(m, bidx, -1.0))
            sc_ = jnp.min(jnp.where(m, cs, jnp.float32(1e9)))
            take = jnp.logical_and(B < 0.0, bc >= 0.0)
            B = jnp.where(take, bc, B)
            S = jnp.where(take, sc_, S)
            carry = jnp.max(cs)
        return B, S

    _SHRINK = (1.0 + 2.0 / 64.0) / NB

    def level(lo, wb, scale):
        zero_hist()
        hist_pass(lo, scale)
        B, S = find()
        return lo + B * wb, S

    def refine(args):
        t, wb, scale, _ = args
        lo2 = t - wb * jnp.float32(1.0 / 64.0)
        wb2 = wb * jnp.float32(_SHRINK)
        scale2 = scale * jnp.float32(1.0 / _SHRINK)
        t2, S2 = level(lo2, wb2, scale2)
        return t2, wb2, scale2, S2

    def row_step(k, carry_in):
        start = carry_in
        row = start + k
        pltpu.sync_copy(sim_ref.at[row], rowbuf)

        wb1 = jnp.float32(1.02 / NB)
        sc1 = jnp.float32(NB / 1.02)
        t1, s1 = level(jnp.float32(-0.01), wb1, sc1)
        st = (t1, wb1, sc1, s1)
        for _ in range(3):
            st = lax.cond(st[3] != jnp.float32(KP1), refine, lambda a: a, st)
        t = st[0]

        def mbody(i, c):
            v = rowbuf[pl.ds(i * NLANE, NLANE)]
            keep = jnp.logical_and(v >= t, v > 0.0)
            outbuf[pl.ds(i * NLANE, NLANE)] = jnp.where(keep, v, 0.0)
            return c
        lax.fori_loop(0, nvreg, mbody, 0)
        pltpu.sync_copy(outbuf, out_ref.at[row])
        return carry_in

    nrows = sim_ref.shape[0]
    per_w = (nrows + NWORKERS - 1) // NWORKERS
    wid = lax.axis_index("s") * 2 + lax.axis_index("c")
    start = wid * per_w
    cnt = jnp.maximum(jnp.minimum(per_w, nrows - start), 0)
    lax.fori_loop(0, cnt, row_step, start)


def kernel(features, w1, w2):
    n, d = features.shape
    w1r = w1.reshape(1, d)
    w2r = w2.reshape(1, d)

    emb = pl.pallas_call(
        _prep_body,
        out_shape=jax.ShapeDtypeStruct((n, d), jnp.float32),
    )(features, w1r, w2r)

    bm = 200 if n % 200 == 0 else n
    grid = n // bm
    sim = pl.pallas_call(
        _mm_body,
        grid=(grid,),
        in_specs=[
            pl.BlockSpec((bm, d), lambda i: (i, 0)),
            pl.BlockSpec((n, d), lambda i: (0, 0)),
        ],
        out_specs=pl.BlockSpec((bm, n), lambda i: (i, 0)),
        out_shape=jax.ShapeDtypeStruct((n, n), jnp.float32),
    )(emb, emb)

    sc_topk = functools.partial(
        pl.kernel,
        out_type=jax.ShapeDtypeStruct((n, n), jnp.float32),
        mesh=plsc.VectorSubcoreMesh(core_axis_name="c", subcore_axis_name="s"),
        compiler_params=pltpu.CompilerParams(needs_layout_passes=False),
        scratch_types=[
            pltpu.VMEM((n,), jnp.float32),
            pltpu.VMEM((n,), jnp.float32),
            pltpu.VMEM((NB * NLANE,), jnp.float32),
        ],
    )(functools.partial(_sc_topk_body, ncol=n))
    return sc_topk(sim)


# speedup vs baseline: 2.5930x; 2.5930x over previous
import functools

import jax
import jax.numpy as jnp
from jax import lax
from jax.experimental import pallas as pl
from jax.experimental.pallas import tpu as pltpu
from jax.experimental.pallas import tpu_sc as plsc

KP1 = 31
NB = 128
NLANE = 16
NWORKERS = 32


def _prep_body(f_ref, w1_ref, w2_ref, emb_ref):
    h = f_ref[...] * w1_ref[...]
    h = jnp.maximum(h, 0.0) * w2_ref[...]
    n2 = jnp.sum(h * h, axis=1, keepdims=True)
    norm = jnp.sqrt(n2)
    emb_ref[...] = h / jnp.maximum(norm, 1e-12)


def _mm_body(a_ref, b_ref, out_ref):
    out_ref[...] = lax.dot_general(
        a_ref[...], b_ref[...], (((1,), (1,)), ((), ())),
        preferred_element_type=jnp.float32)


def _sc_topk_body(sim_ref, out_ref, rowbuf, outbuf, hist, *, ncol):
    nvreg = ncol // NLANE
    lanes = lax.iota(jnp.int32, NLANE)
    lane_lo = (lanes * NB).astype(jnp.float32)
    lane_hi = lane_lo + jnp.float32(NB - 1)
    ones16 = jnp.ones((NLANE,), jnp.float32)
    zeros16 = jnp.zeros((NLANE,), jnp.float32)

    def zero_hist():
        @plsc.parallel_loop(0, NB * NLANE // NLANE, unroll=8)
        def _(j):
            hist[pl.ds(j * NLANE, NLANE)] = zeros16

    def hist_pass(lo, scale):
        off = lane_lo - lo * scale

        @plsc.parallel_loop(0, nvreg, unroll=8)
        def _(i):
            v = rowbuf[pl.ds(i * NLANE, NLANE)]
            x = v * scale + off
            x = jnp.minimum(jnp.maximum(x, lane_lo), lane_hi)
            plsc.addupdate_scatter(hist, [x.astype(jnp.int32)], ones16)

    def find():
        nj = NB // NLANE
        folded = []
        for j in range(nj):
            parts = [hist[pl.ds(j * NLANE + l * NB, NLANE)]
                     for l in range(NLANE)]
            while len(parts) > 1:
                parts = [parts[k] + parts[k + 1]
                         for k in range(0, len(parts) - 1, 2)] + (
                             [parts[-1]] if len(parts) % 2 else [])
            folded.append(parts[0])
        tots = [jnp.max(plsc.cumsum(a)) for a in folded]
        carry = jnp.float32(0.0)
        carries = [None] * nj
        for j in reversed(range(nj)):
            carries[j] = carry
            carry = carry + tots[j]
        B = jnp.float32(-1.0)
        S = jnp.float32(0.0)
        iota_f = lax.iota(jnp.int32, NLANE).astype(jnp.float32)
        for j in reversed(range(nj)):
            cs = plsc.cumsum(lax.rev(folded[j], (0,))) + carries[j]
            m = cs >= jnp.float32(KP1)
            bidx = jnp.float32(NLANE * j + NLANE - 1) - iota_f
            bc = jnp.max(jnp.where(m, bidx, -1.0))
            sc_ = jnp.min(jnp.where(m, cs, jnp.float32(1e9)))
            take = jnp.logical_and(B < 0.0, bc >= 0.0)
            B = jnp.where(take, bc, B)
            S = jnp.where(take, sc_, S)
        return B, S

    _SHRINK = (1.0 + 2.0 / 64.0) / NB

    def level(lo, wb, scale):
        zero_hist()
        hist_pass(lo, scale)
        B, S = find()
        return lo + B * wb, S

    def refine(args):
        t, wb, scale, _ = args
        lo2 = t - wb * jnp.float32(1.0 / 64.0)
        wb2 = wb * jnp.float32(_SHRINK)
        scale2 = scale * jnp.float32(1.0 / _SHRINK)
        t2, S2 = level(lo2, wb2, scale2)
        return t2, wb2, scale2, S2

    def row_step(k, carry_in):
        start = carry_in
        row = start + k
        pltpu.sync_copy(sim_ref.at[row], rowbuf)

        wb1 = jnp.float32(1.02 / NB)
        sc1 = jnp.float32(NB / 1.02)
        t1, s1 = level(jnp.float32(-0.01), wb1, sc1)
        st = (t1, wb1, sc1, s1)
        for _ in range(3):
            st = lax.cond(st[3] != jnp.float32(KP1), refine, lambda a: a, st)
        t = st[0]

        @plsc.parallel_loop(0, nvreg, unroll=8)
        def _(i):
            v = rowbuf[pl.ds(i * NLANE, NLANE)]
            keep = jnp.logical_and(v >= t, v > 0.0)
            outbuf[pl.ds(i * NLANE, NLANE)] = jnp.where(keep, v, 0.0)
        pltpu.sync_copy(outbuf, out_ref.at[row])
        return carry_in

    nrows = sim_ref.shape[0]
    per_w = (nrows + NWORKERS - 1) // NWORKERS
    wid = lax.axis_index("s") * 2 + lax.axis_index("c")
    start = wid * per_w
    cnt = jnp.maximum(jnp.minimum(per_w, nrows - start), 0)
    lax.fori_loop(0, cnt, row_step, start)


def kernel(features, w1, w2):
    n, d = features.shape
    w1r = w1.reshape(1, d)
    w2r = w2.reshape(1, d)

    emb = pl.pallas_call(
        _prep_body,
        out_shape=jax.ShapeDtypeStruct((n, d), jnp.float32),
    )(features, w1r, w2r)

    bm = 200 if n % 200 == 0 else n
    grid = n // bm
    sim = pl.pallas_call(
        _mm_body,
        grid=(grid,),
        in_specs=[
            pl.BlockSpec((bm, d), lambda i: (i, 0)),
            pl.BlockSpec((n, d), lambda i: (0, 0)),
        ],
        out_specs=pl.BlockSpec((bm, n), lambda i: (i, 0)),
        out_shape=jax.ShapeDtypeStruct((n, n), jnp.float32),
    )(emb, emb)

    sc_topk = functools.partial(
        pl.kernel,
        out_type=jax.ShapeDtypeStruct((n, n), jnp.float32),
        mesh=plsc.VectorSubcoreMesh(core_axis_name="c", subcore_axis_name="s"),
        compiler_params=pltpu.CompilerParams(needs_layout_passes=False),
        scratch_types=[
            pltpu.VMEM((n,), jnp.float32),
            pltpu.VMEM((n,), jnp.float32),
            pltpu.VMEM((NB * NLANE,), jnp.float32),
        ],
    )(functools.partial(_sc_topk_body, ncol=n))
    return sc_topk(sim)


# speedup vs baseline: 8.2124x; 3.1671x over previous
import functools

import jax
import jax.numpy as jnp
from jax import lax
from jax.experimental import pallas as pl

KP1 = 31
BISECT_ITERS = 21


def _prep_body(f_ref, w1_ref, w2_ref, emb_ref):
    h = f_ref[...] * w1_ref[...]
    h = jnp.maximum(h, 0.0) * w2_ref[...]
    n2 = jnp.sum(h * h, axis=1, keepdims=True)
    norm = jnp.sqrt(n2)
    emb_ref[...] = h / jnp.maximum(norm, 1e-12)


def _main_body(a_ref, b_ref, out_ref, *, bm):
    a = a_ref[...]
    b = b_ref[...]
    sim = lax.dot_general(a, b, (((1,), (1,)), ((), ())),
                          preferred_element_type=jnp.float32)

    lo = jnp.full((bm, 1), -0.01, jnp.float32)
    hi = jnp.full((bm, 1), 1.01, jnp.float32)

    def body(_, carry):
        lo, hi = carry
        mid = 0.5 * (lo + hi)
        cnt = jnp.sum((sim >= mid).astype(jnp.float32), axis=1, keepdims=True)
        ge = cnt >= KP1
        return jnp.where(ge, mid, lo), jnp.where(ge, hi, mid)

    lo, hi = lax.fori_loop(0, BISECT_ITERS, body, (lo, hi))
    out_ref[...] = jnp.where((sim >= lo) & (sim > 0.0), sim, 0.0)


def kernel(features, w1, w2):
    n, d = features.shape
    w1r = w1.reshape(1, d)
    w2r = w2.reshape(1, d)

    emb = pl.pallas_call(
        _prep_body,
        out_shape=jax.ShapeDtypeStruct((n, d), jnp.float32),
    )(features, w1r, w2r)

    bm = 400 if n % 400 == 0 else n
    grid = n // bm

    out = pl.pallas_call(
        functools.partial(_main_body, bm=bm),
        grid=(grid,),
        in_specs=[
            pl.BlockSpec((bm, d), lambda i: (i, 0)),
            pl.BlockSpec((n, d), lambda i: (0, 0)),
        ],
        out_specs=pl.BlockSpec((bm, n), lambda i: (i, 0)),
        out_shape=jax.ShapeDtypeStruct((n, n), jnp.float32),
    )(emb, emb)
    return out
